# Initial kernel scaffold; baseline (speedup 1.0000x reference)
import functools

import jax
import jax.numpy as jnp
from jax import lax
from jax.experimental import pallas as pl
from jax.experimental.pallas import tpu as pltpu
from jax.experimental.pallas import tpu_sc as plsc

N_NODES = 10000
N_EDGES = 160000
IN_DIM = 500
HID = 8

NC = 2
NS = 16
NW = NC * NS

CHUNK = 128
NCHUNK = 40
EPW = CHUNK * NCHUNK
N_PADROW = 16
NPAD = N_NODES + N_PADROW
SLAB = NPAD // NS


def _tc_linear_body(x_ref, w_ref, b_ref, o_ref):
    o_ref[...] = (
        jnp.dot(x_ref[...], w_ref[...], preferred_element_type=jnp.float32)
        + b_ref[...]
    )


def _tc_linear(x, w, b, m_block):
    grid = (NPAD + m_block - 1) // m_block
    return pl.pallas_call(
        _tc_linear_body,
        grid=(grid,),
        in_specs=[
            pl.BlockSpec((m_block, x.shape[1]), lambda i: (i, 0)),
            pl.BlockSpec((x.shape[1], HID), lambda i: (0, 0)),
            pl.BlockSpec((1, HID), lambda i: (0, 0)),
        ],
        out_specs=pl.BlockSpec((m_block, HID), lambda i: (i, 0)),
        out_shape=jax.ShapeDtypeStruct((NPAD, HID), jnp.float32),
    )(x, w, b)


def _tc_combine_linear_body(p_ref, w_ref, b_ref, o_ref):
    h = jnp.maximum(p_ref[0] + p_ref[1], 0.0)
    o_ref[...] = (
        jnp.dot(h, w_ref[...], preferred_element_type=jnp.float32) + b_ref[...]
    )


def _tc_combine_linear(p, w, b):
    return pl.pallas_call(
        _tc_combine_linear_body,
        grid=(NS,),
        in_specs=[
            pl.BlockSpec((NC, SLAB, HID), lambda i: (0, i, 0)),
            pl.BlockSpec((HID, HID), lambda i: (0, 0)),
            pl.BlockSpec((1, HID), lambda i: (0, 0)),
        ],
        out_specs=pl.BlockSpec((SLAB, HID), lambda i: (i, 0)),
        out_shape=jax.ShapeDtypeStruct((NPAD, HID), jnp.float32),
    )(p, w, b)


def _tc_combine_body(p_ref, o_ref):
    o_ref[...] = p_ref[0] + p_ref[1]


def _tc_combine(p):
    return pl.pallas_call(
        _tc_combine_body,
        grid=(NS,),
        in_specs=[pl.BlockSpec((NC, SLAB, HID), lambda i: (0, i, 0))],
        out_specs=pl.BlockSpec((SLAB, HID), lambda i: (i, 0)),
        out_shape=jax.ShapeDtypeStruct((NPAD, HID), jnp.float32),
    )(p)


def _sc_agg_body(table_hbm, src_hbm, dst_hbm, zeros_hbm, out_hbm,
                 acc_sp, table_sp, src_v, dst_v, chunk_v, sem):
    c = lax.axis_index("c")
    s = lax.axis_index("s")
    wid = c * NS + s
    slab = pl.ds(s * SLAB, SLAB)

    pltpu.sync_copy(zeros_hbm.at[slab], acc_sp.at[slab])
    pltpu.sync_copy(table_hbm.at[slab], table_sp.at[slab])
    pltpu.sync_copy(src_hbm.at[wid], src_v)
    pltpu.sync_copy(dst_hbm.at[wid], dst_v)
    plsc.subcore_barrier()

    def body(j, carry):
        pltpu.async_copy(table_sp.at[src_v.at[j]], chunk_v, sem).wait()
        pltpu.sync_copy(chunk_v, acc_sp.at[dst_v.at[j]], add=True)
        return carry

    lax.fori_loop(0, NCHUNK, body, 0)
    plsc.subcore_barrier()

    pltpu.sync_copy(acc_sp.at[slab], out_hbm.at[c].at[slab])


_sc_agg = functools.partial(
    pl.kernel,
    out_type=jax.ShapeDtypeStruct((NC, NPAD, HID), jnp.float32),
    mesh=plsc.VectorSubcoreMesh(
        core_axis_name="c", subcore_axis_name="s", num_cores=NC,
        num_subcores=NS,
    ),
    scratch_types=[
        pltpu.VMEM_SHARED((NPAD, HID), jnp.float32),
        pltpu.VMEM_SHARED((NPAD, HID), jnp.float32),
        pltpu.VMEM((NCHUNK, CHUNK), jnp.int32),
        pltpu.VMEM((NCHUNK, CHUNK), jnp.int32),
        pltpu.VMEM((CHUNK, HID), jnp.float32),
        pltpu.SemaphoreType.DMA,
    ],
)(_sc_agg_body)


def kernel(x, edge_index, W1, b1, W2, b2):
    src = edge_index[0].astype(jnp.int32)
    dst = edge_index[1].astype(jnp.int32)

    n_pad = NW * EPW - N_EDGES
    pad_idx = N_NODES + (jnp.arange(n_pad, dtype=jnp.int32) % N_PADROW)
    src_p = jnp.concatenate([src, pad_idx]).reshape(NW, NCHUNK, CHUNK)
    dst_p = jnp.concatenate([dst, pad_idx]).reshape(NW, NCHUNK, CHUNK)

    zeros = jnp.zeros((NPAD, HID), jnp.float32)
    w2p = jnp.pad(W2, ((0, 0), (0, HID - W2.shape[1])))
    b1r = b1.reshape(1, HID)
    b2r = jnp.pad(b2, (0, HID - b2.shape[0])).reshape(1, HID)

    support1 = _tc_linear(x, W1, b1r, m_block=SLAB)
    p1 = _sc_agg(support1, src_p, dst_p, zeros)
    support2 = _tc_combine_linear(p1, w2p, b2r)
    p2 = _sc_agg(support2, src_p, dst_p, zeros)
    out = _tc_combine(p2)
    return out[:N_NODES, : W2.shape[1]]


# speedup vs baseline: 9.4043x; 9.4043x over previous
import functools

import jax
import jax.numpy as jnp
from jax import lax
from jax.experimental import pallas as pl
from jax.experimental.pallas import tpu as pltpu
from jax.experimental.pallas import tpu_sc as plsc

N_NODES = 10000
N_EDGES = 160000
IN_DIM = 500
HID = 8

NC = 2
NS = 16
NW = NC * NS

CHUNK = 128
NCHUNK = 40
EPW = CHUNK * NCHUNK
N_PADROW = 112
NPAD = N_NODES + N_PADROW
SLAB = NPAD // NS


def _tc_linear_body(x_ref, w_ref, b_ref, o_ref):
    o_ref[...] = (
        jnp.dot(x_ref[...], w_ref[...], preferred_element_type=jnp.float32)
        + b_ref[...]
    )


def _tc_linear(x, w, b, m_block):
    grid = x.shape[0] // m_block
    return pl.pallas_call(
        _tc_linear_body,
        grid=(grid,),
        in_specs=[
            pl.BlockSpec((m_block, x.shape[1]), lambda i: (i, 0)),
            pl.BlockSpec((x.shape[1], HID), lambda i: (0, 0)),
            pl.BlockSpec((1, HID), lambda i: (0, 0)),
        ],
        out_specs=pl.BlockSpec((m_block, HID), lambda i: (i, 0)),
        out_shape=jax.ShapeDtypeStruct((x.shape[0], HID), jnp.float32),
    )(x, w, b)


def _tc_combine_linear_body(p_ref, w_ref, b_ref, o_ref):
    h = jnp.maximum(p_ref[0] + p_ref[1], 0.0)
    o_ref[...] = (
        jnp.dot(h, w_ref[...], preferred_element_type=jnp.float32) + b_ref[...]
    )


def _tc_combine_linear(p, w, b):
    return pl.pallas_call(
        _tc_combine_linear_body,
        out_shape=jax.ShapeDtypeStruct((NPAD, HID), jnp.float32),
    )(p, w, b)


def _tc_combine_body(p_ref, o_ref):
    o_ref[...] = p_ref[0] + p_ref[1]


def _tc_combine(p):
    return pl.pallas_call(
        _tc_combine_body,
        out_shape=jax.ShapeDtypeStruct((NPAD, HID), jnp.float32),
    )(p)


def _sc_agg_body(table_hbm, src_hbm, dst_hbm, zeros_hbm, out_hbm,
                 acc_sp, src_v, dst_v, chunk_v, sem):
    c = lax.axis_index("c")
    s = lax.axis_index("s")
    wid = c * NS + s
    slab = pl.ds(s * SLAB, SLAB)

    pltpu.sync_copy(zeros_hbm.at[slab], acc_sp.at[slab])
    pltpu.sync_copy(src_hbm.at[wid], src_v)
    pltpu.sync_copy(dst_hbm.at[wid], dst_v)
    plsc.subcore_barrier()

    def body(j, carry):
        pltpu.async_copy(table_hbm.at[src_v.at[j]], chunk_v, sem).wait()
        pltpu.sync_copy(chunk_v, acc_sp.at[dst_v.at[j]], add=True)
        return carry

    lax.fori_loop(0, NCHUNK, body, 0)
    plsc.subcore_barrier()

    pltpu.sync_copy(acc_sp.at[slab], out_hbm.at[c].at[slab])


_sc_agg = functools.partial(
    pl.kernel,
    out_type=jax.ShapeDtypeStruct((NC, NPAD, HID), jnp.float32),
    mesh=plsc.VectorSubcoreMesh(
        core_axis_name="c", subcore_axis_name="s", num_cores=NC,
        num_subcores=NS,
    ),
    compiler_params=pltpu.CompilerParams(use_tc_tiling_on_sc=False),
    scratch_types=[
        pltpu.VMEM_SHARED((NPAD, HID), jnp.float32),
        pltpu.VMEM((NCHUNK, CHUNK), jnp.int32),
        pltpu.VMEM((NCHUNK, CHUNK), jnp.int32),
        pltpu.VMEM((CHUNK, HID), jnp.float32),
        pltpu.SemaphoreType.DMA,
    ],
)(_sc_agg_body)


def kernel(x, edge_index, W1, b1, W2, b2):
    src = edge_index[0].astype(jnp.int32)
    dst = edge_index[1].astype(jnp.int32)

    n_pad = NW * EPW - N_EDGES
    pad_idx = N_NODES + (jnp.arange(n_pad, dtype=jnp.int32) % N_PADROW)
    src_p = jnp.concatenate([src, pad_idx]).reshape(NW, NCHUNK, CHUNK)
    dst_p = jnp.concatenate([dst, pad_idx]).reshape(NW, NCHUNK, CHUNK)

    zeros = jnp.zeros((NPAD, HID), jnp.float32)
    w2p = jnp.pad(W2, ((0, 0), (0, HID - W2.shape[1])))
    b1r = b1.reshape(1, HID)
    b2r = jnp.pad(b2, (0, HID - b2.shape[0])).reshape(1, HID)

    support1 = _tc_linear(x, W1, b1r, m_block=1000)
    support1 = jnp.pad(support1, ((0, NPAD - N_NODES), (0, 0)))
    p1 = _sc_agg(support1, src_p, dst_p, zeros)
    support2 = _tc_combine_linear(p1, w2p, b2r)
    p2 = _sc_agg(support2, src_p, dst_p, zeros)
    out = _tc_combine(p2)
    return out[:N_NODES, : W2.shape[1]]


# speedup vs baseline: 16.7506x; 1.7812x over previous
import functools

import jax
import jax.numpy as jnp
from jax import lax
from jax.experimental import pallas as pl
from jax.experimental.pallas import tpu as pltpu
from jax.experimental.pallas import tpu_sc as plsc

N_NODES = 10000
N_EDGES = 160000
IN_DIM = 500
HID = 8

NC = 2
NS = 16
NW = NC * NS

CHUNK = 128
NCHUNK = 40
NBUF = 4
EPW = CHUNK * NCHUNK
N_PADROW = 112
NPAD = N_NODES + N_PADROW
SLAB = NPAD // NS
NROW128 = NPAD * HID // 128


def _tc_linear_body(xt_ref, w_ref, b_ref, o_ref):
    o_ref[...] = (
        lax.dot_general(
            xt_ref[...], w_ref[...],
            dimension_numbers=(((0,), (0,)), ((), ())),
            preferred_element_type=jnp.float32,
        )
        + b_ref[...]
    )


def _tc_linear(xt, w, b):
    return pl.pallas_call(
        _tc_linear_body,
        out_shape=jax.ShapeDtypeStruct((xt.shape[1], HID), jnp.float32),
    )(xt, w, b)


def _tc_combine_linear_body(p_ref, w_ref, b_ref, o_ref):
    h = jnp.maximum(p_ref[0] + p_ref[1], 0.0)
    o_ref[...] = (
        jnp.dot(h, w_ref[...], preferred_element_type=jnp.float32) + b_ref[...]
    )


def _tc_combine_linear(p, w, b):
    return pl.pallas_call(
        _tc_combine_linear_body,
        out_shape=jax.ShapeDtypeStruct((NROW128, 128), jnp.float32),
    )(p, w, b)


def _tc_combine_body(p_ref, o_ref):
    o_ref[...] = p_ref[0] + p_ref[1]


def _tc_combine(p):
    return pl.pallas_call(
        _tc_combine_body,
        out_shape=jax.ShapeDtypeStruct((NROW128, 128), jnp.float32),
    )(p)


def _sc_agg_body(table_hbm, src_hbm, dst_hbm, zeros_hbm, out_hbm,
                 acc_sp, src_v, dst_v, chunk_v, sem_g, sem_s):
    c = lax.axis_index("c")
    s = lax.axis_index("s")
    wid = c * NS + s
    slab = pl.ds(s * SLAB, SLAB)

    pltpu.sync_copy(zeros_hbm.at[slab], acc_sp.at[slab])
    pltpu.sync_copy(src_hbm.at[wid], src_v)
    pltpu.sync_copy(dst_hbm.at[wid], dst_v)
    plsc.subcore_barrier()

    def gather(j):
        pltpu.async_copy(table_hbm.at[src_v.at[j]],
                         chunk_v.at[lax.rem(j, NBUF)], sem_g)

    def wait_gather(j):
        pltpu.make_async_copy(table_hbm.at[src_v.at[j]],
                              chunk_v.at[lax.rem(j, NBUF)], sem_g).wait()

    def scatter(j):
        pltpu.async_copy(chunk_v.at[lax.rem(j, NBUF)],
                         acc_sp.at[dst_v.at[j]], sem_s, add=True)

    def wait_scatter(j):
        pltpu.make_async_copy(chunk_v.at[lax.rem(j, NBUF)],
                              acc_sp.at[dst_v.at[j]], sem_s).wait()

    for j in range(NBUF - 1):
        gather(j)

    def body(j, carry):
        wait_gather(j)
        scatter(j)
        @pl.when((j >= 1) & (j + NBUF - 1 < NCHUNK))
        def _():
            wait_scatter(j - 1)

        @pl.when(j + NBUF - 1 < NCHUNK)
        def _():
            gather(j + NBUF - 1)

        return carry

    lax.fori_loop(0, NCHUNK, body, 0)
    for j in range(NCHUNK - NBUF, NCHUNK):
        wait_scatter(j)
    plsc.subcore_barrier()

    pltpu.sync_copy(acc_sp.at[slab], out_hbm.at[c].at[slab])


_sc_agg = functools.partial(
    pl.kernel,
    out_type=jax.ShapeDtypeStruct((NC, NPAD, HID), jnp.float32),
    mesh=plsc.VectorSubcoreMesh(
        core_axis_name="c", subcore_axis_name="s", num_cores=NC,
        num_subcores=NS,
    ),
    compiler_params=pltpu.CompilerParams(use_tc_tiling_on_sc=False),
    scratch_types=[
        pltpu.VMEM_SHARED((NPAD, HID), jnp.float32),
        pltpu.VMEM((NCHUNK, CHUNK), jnp.int32),
        pltpu.VMEM((NCHUNK, CHUNK), jnp.int32),
        pltpu.VMEM((NBUF, CHUNK, HID), jnp.float32),
        pltpu.SemaphoreType.DMA,
        pltpu.SemaphoreType.DMA,
    ],
)(_sc_agg_body)


def kernel(x, edge_index, W1, b1, W2, b2):
    n_pad = NW * EPW - N_EDGES
    pad_idx = N_NODES + (jnp.arange(n_pad, dtype=jnp.int32) % N_PADROW)
    ei = jnp.concatenate(
        [edge_index.astype(jnp.int32),
         jnp.broadcast_to(pad_idx, (2, n_pad))], axis=1)
    src_p = ei[0].reshape(NW, NCHUNK, CHUNK)
    dst_p = ei[1].reshape(NW, NCHUNK, CHUNK)

    zeros = jnp.zeros((NPAD, HID), jnp.float32)
    w2p = jnp.pad(W2, ((0, 0), (0, HID - W2.shape[1])))
    w2bd = jnp.kron(jnp.eye(16, dtype=jnp.float32), w2p)
    b1r = b1.reshape(1, HID)
    b2r = jnp.tile(jnp.pad(b2, (0, HID - b2.shape[0])), 16).reshape(1, 128)

    support1 = _tc_linear(x.T, W1, b1r)
    support1 = jnp.pad(support1, ((0, NPAD - N_NODES), (0, 0)))
    p1 = _sc_agg(support1, src_p, dst_p, zeros)

    p1v = p1.reshape(NC, NROW128, 128)
    support2 = _tc_combine_linear(p1v, w2bd, b2r).reshape(NPAD, HID)
    p2 = _sc_agg(support2, src_p, dst_p, zeros)

    out = _tc_combine(p2.reshape(NC, NROW128, 128)).reshape(NPAD, HID)
    return out[:N_NODES, : W2.shape[1]]


# speedup vs baseline: 20.5005x; 1.2239x over previous
import functools

import jax
import jax.numpy as jnp
from jax import lax
from jax.experimental import pallas as pl
from jax.experimental.pallas import tpu as pltpu
from jax.experimental.pallas import tpu_sc as plsc

N_NODES = 10000
N_EDGES = 160000
IN_DIM = 500
HID = 8

NC = 2
NS = 16
NW = NC * NS

CHUNK = 512
NCHUNK = 10
NBUF = 4
EPW = CHUNK * NCHUNK
N_PADROW = 112
NPAD = N_NODES + N_PADROW
SLAB = NPAD // NS
NROW128 = NPAD * HID // 128


def _tc_linear_body(xt_ref, w_ref, b_ref, o_ref):
    o_ref[: xt_ref.shape[1], :] = (
        lax.dot_general(
            xt_ref[...], w_ref[...],
            dimension_numbers=(((0,), (0,)), ((), ())),
            preferred_element_type=jnp.float32,
        )
        + b_ref[...]
    )


def _tc_linear(xt, w, b):
    return pl.pallas_call(
        _tc_linear_body,
        out_shape=jax.ShapeDtypeStruct((NPAD, HID), jnp.float32),
    )(xt, w, b)


def _tc_combine_linear_body(p_ref, w_ref, b_ref, o_ref):
    h = jnp.maximum(p_ref[0] + p_ref[1], 0.0)
    o_ref[...] = (
        jnp.dot(h, w_ref[...], preferred_element_type=jnp.float32) + b_ref[...]
    )


def _tc_combine_linear(p, w, b):
    return pl.pallas_call(
        _tc_combine_linear_body,
        out_shape=jax.ShapeDtypeStruct((NROW128, 128), jnp.float32),
    )(p, w, b)


def _tc_combine_body(p_ref, o_ref):
    o_ref[...] = p_ref[0] + p_ref[1]


def _tc_combine(p):
    return pl.pallas_call(
        _tc_combine_body,
        out_shape=jax.ShapeDtypeStruct((NROW128, 128), jnp.float32),
    )(p)


def _sc_agg_body(table_hbm, src_hbm, dst_hbm, zeros_hbm, out_hbm,
                 acc_sp, src_v, dst_v, chunk_v, sem_g, sem_s):
    c = lax.axis_index("c")
    s = lax.axis_index("s")
    wid = c * NS + s
    slab = pl.ds(s * SLAB, SLAB)

    pltpu.sync_copy(zeros_hbm.at[slab], acc_sp.at[slab])
    pltpu.sync_copy(src_hbm.at[wid], src_v)
    pltpu.sync_copy(dst_hbm.at[wid], dst_v)
    plsc.subcore_barrier()

    def gather(j):
        pltpu.async_copy(table_hbm.at[src_v.at[j]],
                         chunk_v.at[lax.rem(j, NBUF)], sem_g)

    def wait_gather(j):
        pltpu.make_async_copy(table_hbm.at[src_v.at[j]],
                              chunk_v.at[lax.rem(j, NBUF)], sem_g).wait()

    def scatter(j):
        pltpu.async_copy(chunk_v.at[lax.rem(j, NBUF)],
                         acc_sp.at[dst_v.at[j]], sem_s, add=True)

    def wait_scatter(j):
        pltpu.make_async_copy(chunk_v.at[lax.rem(j, NBUF)],
                              acc_sp.at[dst_v.at[j]], sem_s).wait()

    for j in range(NBUF - 1):
        gather(j)

    def body(j, carry):
        wait_gather(j)
        scatter(j)
        @pl.when((j >= 1) & (j + NBUF - 1 < NCHUNK))
        def _():
            wait_scatter(j - 1)

        @pl.when(j + NBUF - 1 < NCHUNK)
        def _():
            gather(j + NBUF - 1)

        return carry

    lax.fori_loop(0, NCHUNK, body, 0)
    for j in range(NCHUNK - NBUF, NCHUNK):
        wait_scatter(j)
    plsc.subcore_barrier()

    pltpu.sync_copy(acc_sp.at[slab], out_hbm.at[c].at[slab])


_sc_agg = functools.partial(
    pl.kernel,
    out_type=jax.ShapeDtypeStruct((NC, NPAD, HID), jnp.float32),
    mesh=plsc.VectorSubcoreMesh(
        core_axis_name="c", subcore_axis_name="s", num_cores=NC,
        num_subcores=NS,
    ),
    compiler_params=pltpu.CompilerParams(use_tc_tiling_on_sc=False),
    scratch_types=[
        pltpu.VMEM_SHARED((NPAD, HID), jnp.float32),
        pltpu.VMEM((NCHUNK, CHUNK), jnp.int32),
        pltpu.VMEM((NCHUNK, CHUNK), jnp.int32),
        pltpu.VMEM((NBUF, CHUNK, HID), jnp.float32),
        pltpu.SemaphoreType.DMA,
        pltpu.SemaphoreType.DMA,
    ],
)(_sc_agg_body)


def kernel(x, edge_index, W1, b1, W2, b2):
    n_pad = NW * EPW - N_EDGES
    pad_idx = N_NODES + (jnp.arange(n_pad, dtype=jnp.int32) % N_PADROW)
    ei = jnp.concatenate(
        [edge_index.astype(jnp.int32),
         jnp.broadcast_to(pad_idx, (2, n_pad))], axis=1)
    src_p = ei[0].reshape(NW, NCHUNK, CHUNK)
    dst_p = ei[1].reshape(NW, NCHUNK, CHUNK)

    zeros = jnp.zeros((NPAD, HID), jnp.float32)
    w2p = jnp.pad(W2, ((0, 0), (0, HID - W2.shape[1])))
    w2bd = jnp.kron(jnp.eye(16, dtype=jnp.float32), w2p)
    b1r = b1.reshape(1, HID)
    b2r = jnp.tile(jnp.pad(b2, (0, HID - b2.shape[0])), 16).reshape(1, 128)

    support1 = _tc_linear(x.T, W1, b1r)
    p1 = _sc_agg(support1, src_p, dst_p, zeros)

    p1v = p1.reshape(NC, NROW128, 128)
    support2 = _tc_combine_linear(p1v, w2bd, b2r).reshape(NPAD, HID)
    p2 = _sc_agg(support2, src_p, dst_p, zeros)

    out = _tc_combine(p2.reshape(NC, NROW128, 128)).reshape(NPAD, HID)
    return out[:N_NODES, : W2.shape[1]]
